# baseline (speedup 1.0000x reference)
import functools

import jax
import jax.numpy as jnp
from jax import lax
from jax.experimental import pallas as pl
from jax.experimental.pallas import tpu as pltpu
from jax.experimental.pallas import tpu_sc as plsc

N = 6_400_000
S = 4096
NW = 32
LANES = 16
PANEL = 128
NPANEL = N // PANEL
PPC = NPANEL // 2
PBASE = 1560
CPAN = 128
CATOM = CPAN * PANEL
NCHUNK = 13
NSAMP = NCHUNK * 128
WAVE = 128
NWAVE = 13
DLSZ = 1600

_mesh = plsc.VectorSubcoreMesh(core_axis_name="c", subcore_axis_name="s")


@functools.partial(
    pl.kernel,
    out_type=jax.ShapeDtypeStruct((NW, S), jnp.float32),
    mesh=_mesh,
    compiler_params=pltpu.CompilerParams(needs_layout_passes=False),
    scratch_types=[
        pltpu.VMEM((CATOM,), jnp.float32),
        pltpu.VMEM((CATOM,), jnp.float32),
        pltpu.VMEM((NSAMP,), jnp.int32),
        pltpu.VMEM((NSAMP,), jnp.int32),
        pltpu.VMEM((DLSZ,), jnp.float32),
        pltpu.VMEM((DLSZ,), jnp.int32),
        pltpu.VMEM((WAVE * PANEL,), jnp.float32),
        pltpu.VMEM((WAVE * PANEL,), jnp.int32),
        pltpu.VMEM((S,), jnp.float32),
        pltpu.SemaphoreType.DMA,
        pltpu.SemaphoreType.DMA,
        pltpu.SemaphoreType.DMA,
    ],
)
def _sc_segsum(e_hbm, b_hbm, out_hbm,
               v0, v1, sampled, sidx, psum, dlist, vwave, iwave,
               accum, sem0, sem1, semg):
    cid = lax.axis_index("c")
    sid = lax.axis_index("s")
    wid = cid * LANES + sid
    p0 = cid * PPC + sid * PBASE + jnp.minimum(sid, 5) * 8
    np_t = PBASE + jnp.where(sid < 5, 8, 0)

    iota16 = lax.iota(jnp.int32, LANES)
    zeros16f = jnp.zeros((LANES,), jnp.float32)

    def _zero(i, carry):
        accum[pl.ds(i * LANES, LANES)] = zeros16f
        return carry

    lax.fori_loop(0, S // LANES, _zero, 0)

    def _bsi(g, carry):
        i = g * LANES + iota16
        idx = (p0 + i) * PANEL - 1
        idx = jnp.clip(idx, 0, N - 1)
        sidx[pl.ds(g * LANES, LANES)] = idx
        return carry

    lax.fori_loop(0, NSAMP // LANES, _bsi, 0)

    samp_copies = []
    for g in range(NCHUNK):
        samp_copies.append(pltpu.async_copy(
            b_hbm.at[sidx.at[pl.ds(g * 128, 128)]],
            sampled.at[pl.ds(g * 128, 128)], semg))

    vbufs = (v0, v1)
    sems = (sem0, sem1)

    def _chunk_start(k):
        if k < NCHUNK - 1:
            return pl.multiple_of((p0 + k * CPAN) * PANEL, PANEL)
        return pl.multiple_of((p0 + np_t - CPAN) * PANEL, PANEL)

    def _chunk_pbase(k):
        if k < NCHUNK - 1:
            return k * CPAN
        return pl.multiple_of(np_t - CPAN, 8)

    def _issue_v(k):
        slot = k % 2
        return pltpu.async_copy(
            e_hbm.at[pl.ds(_chunk_start(k), CATOM)], vbufs[slot], sems[slot])

    pending = {0: _issue_v(0), 1: _issue_v(1)}

    for k in range(NCHUNK):
        slot = k % 2
        pending.pop(k).wait()
        vb = vbufs[slot]
        lb = _chunk_pbase(k)

        def _panel_group(g, carry):
            inb = (g * LANES + iota16) * PANEL

            def _acc(t, rails):
                sa, sb, sc, sd = rails
                xa = plsc.load_gather(vb, [inb + t])
                xb = plsc.load_gather(vb, [inb + (32 + t)])
                xc = plsc.load_gather(vb, [inb + (64 + t)])
                xd = plsc.load_gather(vb, [inb + (96 + t)])
                return sa + xa, sb + xb, sc + xc, sd + xd

            sa, sb, sc, sd = plsc.parallel_loop(
                0, 32, carry=(zeros16f, zeros16f, zeros16f, zeros16f),
                unroll=4)(_acc)
            ps = (sa + sb) + (sc + sd)
            psum[pl.ds(lb + g * LANES, LANES)] = ps
            return carry

        lax.fori_loop(0, CPAN // LANES, _panel_group, 0)

        if k + 2 < NCHUNK:
            pending[k + 2] = _issue_v(k + 2)

    for c in samp_copies:
        c.wait()

    def _classify(g, off):
        li = g * LANES + iota16
        valid = li < np_t
        s_prev = plsc.load_gather(sampled, [li])
        s_cur = plsc.load_gather(sampled, [li + 1])
        dirty = valid & ((s_prev != s_cur) | (li == 0))
        clean = valid & jnp.logical_not(dirty)
        ps = plsc.load_gather(psum, [li])
        plsc.addupdate_scatter(accum, [s_cur], ps, mask=clean)
        di = dirty.astype(jnp.int32)
        pos = off + plsc.cumsum(di) - 1
        plsc.store_scatter(dlist, [pos], li, mask=dirty)
        return off + jnp.sum(di)

    d_cnt = lax.fori_loop(0, 98, _classify, jnp.int32(0))

    for w in range(NWAVE):
        @pl.when(w * WAVE < d_cnt)
        def _wave():
            def _fetch_group(rg, carry):
                j = w * WAVE + rg * LANES + iota16
                j = jnp.minimum(j, d_cnt - 1)
                pan = p0 + plsc.load_gather(dlist, [j])
                atom0 = pan * PANEL
                for r in range(LANES):
                    off = pl.multiple_of(
                        jnp.sum(jnp.where(iota16 == r, atom0, 0)), PANEL)
                    row = pl.multiple_of((rg * LANES + r) * PANEL, PANEL)
                    pltpu.async_copy(e_hbm.at[pl.ds(off, PANEL)],
                                     vwave.at[pl.ds(row, PANEL)], semg)
                    pltpu.async_copy(b_hbm.at[pl.ds(off, PANEL)],
                                     iwave.at[pl.ds(row, PANEL)], semg)
                return carry

            lax.fori_loop(0, WAVE // LANES, _fetch_group, 0)
            pltpu.make_async_copy(e_hbm.at[pl.ds(0, WAVE * PANEL)],
                                  vwave, semg).wait()
            pltpu.make_async_copy(b_hbm.at[pl.ds(0, WAVE * PANEL)],
                                  iwave, semg).wait()

            def _rowgroup(rg, carry):
                j = w * WAVE + rg * LANES + iota16
                valid = j < d_cnt
                rowb = (rg * LANES + iota16) * PANEL
                cur0 = plsc.load_gather(iwave, [rowb])

                def _estep(t, ec):
                    cur, run = ec
                    x = plsc.load_gather(vwave, [rowb + t])
                    b = plsc.load_gather(iwave, [rowb + t])
                    flush = (b != cur) & valid
                    plsc.addupdate_scatter(accum, [cur], run, mask=flush)
                    run = jnp.where(b != cur, x, run + x)
                    return b, run

                cur_f, run_f = plsc.parallel_loop(
                    0, PANEL, carry=(cur0, zeros16f), unroll=4)(_estep)
                plsc.addupdate_scatter(accum, [cur_f], run_f, mask=valid)
                return carry

            lax.fori_loop(0, WAVE // LANES, _rowgroup, 0)

    pltpu.sync_copy(accum, out_hbm.at[wid])


def _combine_body(p_ref, o_ref):
    o_ref[...] = jnp.sum(p_ref[...], axis=0, keepdims=True)


def kernel(atomic_energy, batch):
    e = atomic_energy.reshape(N)
    b = batch.astype(jnp.int32)
    partials = _sc_segsum(e, b)
    out = pl.pallas_call(
        _combine_body,
        out_shape=jax.ShapeDtypeStruct((1, S), jnp.float32),
    )(partials)
    return out.reshape(S, 1)


# speedup vs baseline: 2.3328x; 2.3328x over previous
import functools

import jax
import jax.numpy as jnp
from jax import lax
from jax.experimental import pallas as pl
from jax.experimental.pallas import tpu as pltpu
from jax.experimental.pallas import tpu_sc as plsc

N = 6_400_000
S = 4096
NW = 32
LANES = 16
PANEL = 128
NPANEL = N // PANEL
PPC = NPANEL // 2
PBASE = 1560
CPAN = 128
CATOM = CPAN * PANEL
NCHUNK = 13
NSAMP = NCHUNK * 128
WAVE = 128
NWAVE = 13
DLSZ = 1600
FSTRIDE = 136
SSTRIDE = 129

_mesh = plsc.VectorSubcoreMesh(core_axis_name="c", subcore_axis_name="s")


@functools.partial(
    pl.kernel,
    out_type=jax.ShapeDtypeStruct((NW, S), jnp.float32),
    mesh=_mesh,
    compiler_params=pltpu.CompilerParams(needs_layout_passes=False),
    scratch_types=[
        pltpu.VMEM((CATOM,), jnp.float32),
        pltpu.VMEM((CATOM,), jnp.float32),
        pltpu.VMEM((NSAMP,), jnp.int32),
        pltpu.VMEM((NSAMP,), jnp.int32),
        pltpu.VMEM((DLSZ,), jnp.float32),
        pltpu.VMEM((DLSZ,), jnp.int32),
        pltpu.VMEM((WAVE * FSTRIDE,), jnp.float32),
        pltpu.VMEM((WAVE * FSTRIDE,), jnp.int32),
        pltpu.VMEM((WAVE * SSTRIDE + 16,), jnp.float32),
        pltpu.VMEM((WAVE * SSTRIDE + 16,), jnp.int32),
        pltpu.VMEM((S,), jnp.float32),
        pltpu.SemaphoreType.DMA,
        pltpu.SemaphoreType.DMA,
        pltpu.SemaphoreType.DMA,
    ],
)
def _sc_segsum(e_hbm, b_hbm, out_hbm,
               v0, v1, sampled, sidx, psum, dlist, vwave, iwave,
               vshuf, ishuf, accum, sem0, sem1, semg):
    cid = lax.axis_index("c")
    sid = lax.axis_index("s")
    wid = cid * LANES + sid
    p0 = cid * PPC + sid * PBASE + jnp.minimum(sid, 5) * 8
    np_t = PBASE + jnp.where(sid < 5, 8, 0)

    iota16 = lax.iota(jnp.int32, LANES)
    zeros16f = jnp.zeros((LANES,), jnp.float32)

    def _zero(i, carry):
        accum[pl.ds(i * LANES, LANES)] = zeros16f
        return carry

    lax.fori_loop(0, S // LANES, _zero, 0)

    def _bsi(g, carry):
        i = g * LANES + iota16
        idx = (p0 + i) * PANEL - 1
        idx = jnp.clip(idx, 0, N - 1)
        sidx[pl.ds(g * LANES, LANES)] = idx
        return carry

    lax.fori_loop(0, NSAMP // LANES, _bsi, 0)

    samp_copies = []
    for g in range(NCHUNK):
        samp_copies.append(pltpu.async_copy(
            b_hbm.at[sidx.at[pl.ds(g * 128, 128)]],
            sampled.at[pl.ds(g * 128, 128)], semg))

    vbufs = (v0, v1)
    sems = (sem0, sem1)

    def _chunk_start(k):
        if k < NCHUNK - 1:
            return pl.multiple_of((p0 + k * CPAN) * PANEL, PANEL)
        return pl.multiple_of((p0 + np_t - CPAN) * PANEL, PANEL)

    def _chunk_pbase(k):
        if k < NCHUNK - 1:
            return k * CPAN
        return pl.multiple_of(np_t - CPAN, 8)

    def _issue_v(k):
        slot = k % 2
        return pltpu.async_copy(
            e_hbm.at[pl.ds(_chunk_start(k), CATOM)], vbufs[slot], sems[slot])

    pending = {0: _issue_v(0), 1: _issue_v(1)}

    for k in range(NCHUNK):
        slot = k % 2
        pending.pop(k).wait()
        vb = vbufs[slot]
        lb = _chunk_pbase(k)

        def _panel_group(g, carry):
            inb = (g * LANES + iota16) * PANEL

            def _acc(t, rails):
                sa, sb, sc, sd = rails
                ts = inb + ((t + iota16) & 31)
                xa = plsc.load_gather(vb, [ts])
                xb = plsc.load_gather(vb, [ts + 32])
                xc = plsc.load_gather(vb, [ts + 64])
                xd = plsc.load_gather(vb, [ts + 96])
                return sa + xa, sb + xb, sc + xc, sd + xd

            sa, sb, sc, sd = plsc.parallel_loop(
                0, 32, carry=(zeros16f, zeros16f, zeros16f, zeros16f),
                unroll=4)(_acc)
            ps = (sa + sb) + (sc + sd)
            psum[pl.ds(lb + g * LANES, LANES)] = ps
            return carry

        lax.fori_loop(0, CPAN // LANES, _panel_group, 0)

        if k + 2 < NCHUNK:
            pending[k + 2] = _issue_v(k + 2)

    for c in samp_copies:
        c.wait()

    def _classify(g, off):
        li = g * LANES + iota16
        valid = li < np_t
        s_prev = plsc.load_gather(sampled, [li])
        s_cur = plsc.load_gather(sampled, [li + 1])
        dirty = valid & ((s_prev != s_cur) | (li == 0))
        clean = valid & jnp.logical_not(dirty)
        ps = plsc.load_gather(psum, [li])
        plsc.addupdate_scatter(accum, [s_cur], ps, mask=clean)
        di = dirty.astype(jnp.int32)
        pos = off + plsc.cumsum(di) - 1
        plsc.store_scatter(dlist, [pos], li, mask=dirty)
        return off + jnp.sum(di)

    d_cnt = lax.fori_loop(0, 98, _classify, jnp.int32(0))

    for w in range(NWAVE):
        @pl.when(w * WAVE < d_cnt)
        def _wave():
            def _fetch_group(rg, carry):
                j = w * WAVE + rg * LANES + iota16
                j = jnp.minimum(j, d_cnt - 1)
                pan = p0 + plsc.load_gather(dlist, [j])
                atom0 = pan * PANEL
                for r in range(LANES):
                    off = pl.multiple_of(
                        jnp.sum(jnp.where(iota16 == r, atom0, 0)), PANEL)
                    row = pl.multiple_of((rg * LANES + r) * FSTRIDE, 8)
                    pltpu.async_copy(e_hbm.at[pl.ds(off, PANEL)],
                                     vwave.at[pl.ds(row, PANEL)], semg)
                    pltpu.async_copy(b_hbm.at[pl.ds(off, PANEL)],
                                     iwave.at[pl.ds(row, PANEL)], semg)
                return carry

            lax.fori_loop(0, WAVE // LANES, _fetch_group, 0)
            pltpu.make_async_copy(e_hbm.at[pl.ds(0, WAVE * PANEL)],
                                  vwave.at[pl.ds(0, WAVE * PANEL)],
                                  semg).wait()
            pltpu.make_async_copy(b_hbm.at[pl.ds(0, WAVE * PANEL)],
                                  iwave.at[pl.ds(0, WAVE * PANEL)],
                                  semg).wait()

            def _shuffle(r, carry):
                src = pl.multiple_of(r * FSTRIDE, 8)
                for jj in range(PANEL // LANES):
                    xv = vwave[pl.ds(src + jj * LANES, LANES)]
                    iv = iwave[pl.ds(src + jj * LANES, LANES)]
                    didx = r * SSTRIDE + jj * LANES + iota16
                    plsc.store_scatter(vshuf, [didx], xv)
                    plsc.store_scatter(ishuf, [didx], iv)
                return carry

            lax.fori_loop(0, WAVE, _shuffle, 0)

            def _rowgroup(rg, carry):
                j = w * WAVE + rg * LANES + iota16
                valid = j < d_cnt
                rowb = (rg * LANES + iota16) * SSTRIDE
                cur0 = plsc.load_gather(ishuf, [rowb])

                def _estep(t, ec):
                    cur, run = ec
                    x = plsc.load_gather(vshuf, [rowb + t])
                    b = plsc.load_gather(ishuf, [rowb + t])
                    flush = (b != cur) & valid
                    plsc.addupdate_scatter(accum, [cur], run, mask=flush)
                    run = jnp.where(b != cur, x, run + x)
                    return b, run

                cur_f, run_f = plsc.parallel_loop(
                    0, PANEL, carry=(cur0, zeros16f), unroll=4)(_estep)
                plsc.addupdate_scatter(accum, [cur_f], run_f, mask=valid)
                return carry

            lax.fori_loop(0, WAVE // LANES, _rowgroup, 0)

    pltpu.sync_copy(accum, out_hbm.at[wid])


def _combine_body(p_ref, o_ref):
    o_ref[...] = jnp.sum(p_ref[...], axis=0, keepdims=True)


def kernel(atomic_energy, batch):
    e = atomic_energy.reshape(N)
    b = batch.astype(jnp.int32)
    partials = _sc_segsum(e, b)
    out = pl.pallas_call(
        _combine_body,
        out_shape=jax.ShapeDtypeStruct((1, S), jnp.float32),
    )(partials)
    return out.reshape(S, 1)
